# Initial kernel scaffold; baseline (speedup 1.0000x reference)
import jax
import jax.numpy as jnp
from jax.experimental import pallas as pl


def kernel(x, edge_index, W, b, gamma, beta):
    raise NotImplementedError("write your pallas kernel here")


# speedup vs baseline: 21.9984x; 21.9984x over previous
import jax
import jax.numpy as jnp
from jax import lax
from jax.experimental import pallas as pl
from jax.experimental.pallas import tpu as pltpu
from jax.experimental.pallas import tpu_sc as plsc

_N = 10000
_E = 160000
_D_IN = 256
_D_OUT = 64
_K = 10
_EPS = 1e-5

_NC = 2
_NS = 16
_L = 16
_NPAD = 10240
_RPT = _NPAD // _NS
_DH = _D_OUT // _NC
_CHUNK = 128
_CH = 79
_EPT = _CH * _CHUNK
_EPAD = _EPT * _NS

_MMBLK = 2048


def _mm_body(x_ref, w_ref, b_ref, o_ref):
    i = pl.program_id(0)
    acc = lax.dot_general(
        x_ref[...], w_ref[...], (((1,), (1,)), ((), ())),
        preferred_element_type=jnp.float32)
    rowid = i * _MMBLK + lax.broadcasted_iota(jnp.int32, (_MMBLK, _DH), 0)
    o_ref[...] = jnp.where(rowid < _N, acc + b_ref[0], 0.0)[None]


def _nrsqrt(x):
    i = lax.bitcast_convert_type(x, jnp.int32)
    i = jnp.int32(0x5F3759DF) - (i >> 1)
    y = lax.bitcast_convert_type(i, jnp.float32)
    for _ in range(3):
        y = y * (jnp.float32(1.5) - jnp.float32(0.5) * x * y * y)
    return y


def _sload(ref, i):
    return ref[pl.ds(i, _L)][0]


_CPB = _RPT // _CHUNK


def _sc_body(h0s, rows_h, cols_h, gb_h, out_h,
             g_s, s_s, deg_s, bn_s,
             rowi, coli, buf, zbuf, ga, g0p, degb, d2b, idb, onesb,
             bnw, bnl, gbl):
    cid = lax.axis_index("c")
    sid = lax.axis_index("s")
    r0 = sid * _RPT

    pltpu.sync_copy(rows_h.at[sid], rowi)
    pltpu.sync_copy(cols_h.at[sid], coli)
    pltpu.sync_copy(gb_h.at[cid], gbl)

    zv = jnp.zeros((_L,), jnp.float32)

    def zrow(i, _):
        zbuf[i, pl.ds(0, _L)] = zv
        zbuf[i, pl.ds(_L, _L)] = zv
        return 0
    lax.fori_loop(0, _CHUNK, zrow, 0)

    def ztmp(t, _):
        degb[pl.ds(t * _L, _L)] = zv
        return 0
    lax.fori_loop(0, _RPT // _L, ztmp, 0)

    def orow(t, _):
        onesb[pl.ds(t * _L, _L)] = jnp.ones((_L,), jnp.float32)
        return 0
    lax.fori_loop(0, _CHUNK // _L, orow, 0)

    pltpu.sync_copy(degb.at[pl.ds(0, _RPT)], deg_s.at[pl.ds(r0, _RPT)])

    def zs(c, _):
        pltpu.sync_copy(zbuf, s_s.at[pl.ds(r0 + c * _CHUNK, _CHUNK)])
        return 0
    lax.fori_loop(0, _CPB, zs, 0)
    plsc.subcore_barrier()

    def degch(j, _):
        pltpu.sync_copy(onesb, deg_s.at[coli.at[j]], add=True)
        return 0
    lax.fori_loop(0, _CH, degch, 0)
    plsc.subcore_barrier()

    pltpu.sync_copy(deg_s.at[pl.ds(r0, _RPT)], degb.at[pl.ds(0, _RPT)])

    def disrow(t, _):
        d = degb[pl.ds(t * _L, _L)] + jnp.float32(1.0)
        y = _nrsqrt(d)
        degb[pl.ds(t * _L, _L)] = y
        d2b[pl.ds(t * _L, _L)] = jnp.float32(0.9) / d
        idb[pl.ds(t * _L, _L)] = d * y
        return 0
    lax.fori_loop(0, _RPT // _L, disrow, 0)

    def initch(c, _):
        pltpu.sync_copy(h0s.at[cid, pl.ds(r0 + c * _CHUNK, _CHUNK)], buf)

        def initrow(i, _):
            r = c * _CHUNK + i
            dis = _sload(degb, r)
            v0 = buf[i, pl.ds(0, _L)] * dis
            v1 = buf[i, pl.ds(_L, _L)] * dis
            ga[r, pl.ds(0, _L)] = v0
            ga[r, pl.ds(_L, _L)] = v1
            g0p[r, pl.ds(0, _L)] = v0 * jnp.float32(0.1)
            g0p[r, pl.ds(_L, _L)] = v1 * jnp.float32(0.1)
            return 0
        lax.fori_loop(0, _CHUNK, initrow, 0)
        return 0
    lax.fori_loop(0, _CPB, initch, 0)
    pltpu.sync_copy(ga, g_s.at[pl.ds(r0, _RPT)])
    plsc.subcore_barrier()

    def kstep(k, _):
        def ech(j, _):
            pltpu.sync_copy(g_s.at[rowi.at[j]], buf)
            pltpu.sync_copy(buf, s_s.at[coli.at[j]], add=True)
            return 0
        lax.fori_loop(0, _CH, ech, 0)
        plsc.subcore_barrier()

        def dchunk(c, _):
            base = c * _CHUNK
            pltpu.sync_copy(s_s.at[pl.ds(r0 + base, _CHUNK)], buf)
            pltpu.sync_copy(zbuf, s_s.at[pl.ds(r0 + base, _CHUNK)])

            def drow(i, _):
                r = base + i
                d2 = _sload(d2b, r)
                v0 = (buf[i, pl.ds(0, _L)] + ga[r, pl.ds(0, _L)]) * d2 \
                    + g0p[r, pl.ds(0, _L)]
                v1 = (buf[i, pl.ds(_L, _L)] + ga[r, pl.ds(_L, _L)]) * d2 \
                    + g0p[r, pl.ds(_L, _L)]
                ga[r, pl.ds(0, _L)] = v0
                ga[r, pl.ds(_L, _L)] = v1
                return 0
            lax.fori_loop(0, _CHUNK, drow, 0)
            return 0
        lax.fori_loop(0, _CPB, dchunk, 0)
        pltpu.sync_copy(ga, g_s.at[pl.ds(r0, _RPT)])
        plsc.subcore_barrier()
        return 0
    lax.fori_loop(0, _K, kstep, 0)

    def bnrow(i, carry):
        s0, s1, q0, q1 = carry
        inv = _sload(idb, i)
        h0v = ga[i, pl.ds(0, _L)] * inv
        h1v = ga[i, pl.ds(_L, _L)] * inv
        return (s0 + h0v, s1 + h1v, q0 + h0v * h0v, q1 + h1v * h1v)
    s0, s1, q0, q1 = lax.fori_loop(0, _RPT, bnrow, (zv, zv, zv, zv))
    bnw[pl.ds(0, _L)] = s0
    bnw[pl.ds(_L, _L)] = s1
    bnw[pl.ds(2 * _L, _L)] = q0
    bnw[pl.ds(3 * _L, _L)] = q1
    pltpu.sync_copy(bnw, bn_s.at[sid])
    plsc.subcore_barrier()
    pltpu.sync_copy(bn_s, bnl)

    def red(t, carry):
        s0, s1, q0, q1 = carry
        return (s0 + bnl[t, pl.ds(0, _L)],
                s1 + bnl[t, pl.ds(_L, _L)],
                q0 + bnl[t, pl.ds(2 * _L, _L)],
                q1 + bnl[t, pl.ds(3 * _L, _L)])
    s0, s1, q0, q1 = lax.fori_loop(0, _NS, red, (zv, zv, zv, zv))
    ninv = jnp.float32(1.0 / _N)
    m0 = s0 * ninv
    m1 = s1 * ninv
    v0 = q0 * ninv - m0 * m0
    v1 = q1 * ninv - m1 * m1
    sc0 = gbl[0, pl.ds(0, _L)] * _nrsqrt(v0 + jnp.float32(_EPS))
    sc1 = gbl[0, pl.ds(_L, _L)] * _nrsqrt(v1 + jnp.float32(_EPS))
    sh0 = gbl[1, pl.ds(0, _L)] - sc0 * m0
    sh1 = gbl[1, pl.ds(_L, _L)] - sc1 * m1

    def arow(i, _):
        inv = _sload(idb, i)
        ga[i, pl.ds(0, _L)] = (ga[i, pl.ds(0, _L)] * inv) * sc0 + sh0
        ga[i, pl.ds(_L, _L)] = (ga[i, pl.ds(_L, _L)] * inv) * sc1 + sh1
        return 0
    lax.fori_loop(0, _RPT, arow, 0)
    pltpu.sync_copy(ga, out_h.at[cid, pl.ds(r0, _RPT)])


def kernel(x, edge_index, W, b, gamma, beta):
    f32 = jnp.float32
    x_pad = jnp.pad(x, ((0, _NPAD - _N), (0, 0)))
    b2 = b.reshape(_NC, 1, _DH)

    h0s = pl.pallas_call(
        _mm_body,
        grid=(_NPAD // _MMBLK, _NC),
        in_specs=[
            pl.BlockSpec((_MMBLK, _D_IN), lambda i, c: (i, 0)),
            pl.BlockSpec((_DH, _D_IN), lambda i, c: (c, 0)),
            pl.BlockSpec((1, 1, _DH), lambda i, c: (c, 0, 0)),
        ],
        out_specs=pl.BlockSpec((1, _MMBLK, _DH), lambda i, c: (c, i, 0)),
        out_shape=jax.ShapeDtypeStruct((_NC, _NPAD, _DH), f32),
    )(x_pad, W, b2)

    padn = _EPAD - _E
    pr = _N + (jnp.arange(padn, dtype=jnp.int32) % 112)
    pc = _N + 112 + (jnp.arange(padn, dtype=jnp.int32) % 112)
    rows_p = jnp.concatenate([edge_index[0], pr]).reshape(_NS, _CH, _CHUNK)
    cols_p = jnp.concatenate([edge_index[1], pc]).reshape(_NS, _CH, _CHUNK)
    gb = jnp.stack([gamma.reshape(_NC, _DH), beta.reshape(_NC, _DH)], axis=1)

    mesh = plsc.VectorSubcoreMesh(core_axis_name="c", subcore_axis_name="s")
    out = pl.kernel(
        _sc_body,
        out_type=jax.ShapeDtypeStruct((_NC, _NPAD, _DH), f32),
        mesh=mesh,
        compiler_params=pltpu.CompilerParams(use_tc_tiling_on_sc=False),
        scratch_types=[
            pltpu.VMEM_SHARED((_NPAD, _DH), f32),
            pltpu.VMEM_SHARED((_NPAD, _DH), f32),
            pltpu.VMEM_SHARED((_NPAD,), f32),
            pltpu.VMEM_SHARED((_NS, 4 * _L), f32),
            pltpu.VMEM((_CH, _CHUNK), jnp.int32),
            pltpu.VMEM((_CH, _CHUNK), jnp.int32),
            pltpu.VMEM((_CHUNK, _DH), f32),
            pltpu.VMEM((_CHUNK, _DH), f32),
            pltpu.VMEM((_RPT, _DH), f32),
            pltpu.VMEM((_RPT, _DH), f32),
            pltpu.VMEM((_RPT + _L,), f32),
            pltpu.VMEM((_RPT + _L,), f32),
            pltpu.VMEM((_RPT + _L,), f32),
            pltpu.VMEM((_CHUNK,), f32),
            pltpu.VMEM((4 * _L,), f32),
            pltpu.VMEM((_NS, 4 * _L), f32),
            pltpu.VMEM((2, _DH), f32),
        ],
    )(h0s, rows_p, cols_p, gb)

    return jnp.concatenate([out[0, :_N], out[1, :_N]], axis=1)


# speedup vs baseline: 28.3190x; 1.2873x over previous
import jax
import jax.numpy as jnp
from jax import lax
from jax.experimental import pallas as pl
from jax.experimental.pallas import tpu as pltpu
from jax.experimental.pallas import tpu_sc as plsc

_N = 10000
_E = 160000
_D_IN = 256
_D_OUT = 64
_K = 10
_EPS = 1e-5

_NC = 2
_NS = 16
_L = 16
_NPAD = 10240
_RPT = _NPAD // _NS
_DH = _D_OUT // _NC
_CHUNK = 128
_CH = 79
_EPT = _CH * _CHUNK
_EPAD = _EPT * _NS

_MMBLK = 2048


def _mm_body(x_ref, w_ref, b_ref, o_ref):
    i = pl.program_id(0)
    acc = lax.dot_general(
        x_ref[...], w_ref[...], (((1,), (1,)), ((), ())),
        preferred_element_type=jnp.float32)
    rowid = i * _MMBLK + lax.broadcasted_iota(jnp.int32, (_MMBLK, _DH), 0)
    o_ref[...] = jnp.where(rowid < _N, acc + b_ref[0], 0.0)[None]


def _nrsqrt(x):
    i = lax.bitcast_convert_type(x, jnp.int32)
    i = jnp.int32(0x5F3759DF) - (i >> 1)
    y = lax.bitcast_convert_type(i, jnp.float32)
    for _ in range(3):
        y = y * (jnp.float32(1.5) - jnp.float32(0.5) * x * y * y)
    return y


def _sload(ref, i):
    return ref[pl.ds(i, _L)][0]


_CPB = _RPT // _CHUNK


def _sc_body(h0s, rows_h, cols_h, gb_h, out_h,
             g_s, s_s, deg_s, bn_s,
             rowi, coli, buf, zbuf, ga, g0p, degb, d2b, idb, onesb,
             bnw, bnl, gbl, gsem, ssem):
    cid = lax.axis_index("c")
    sid = lax.axis_index("s")
    r0 = sid * _RPT

    pltpu.sync_copy(rows_h.at[sid], rowi)
    pltpu.sync_copy(cols_h.at[sid], coli)
    pltpu.sync_copy(gb_h.at[cid], gbl)

    zv = jnp.zeros((_L,), jnp.float32)

    def zrow(i, _):
        zbuf[i, pl.ds(0, _L)] = zv
        zbuf[i, pl.ds(_L, _L)] = zv
        return 0
    lax.fori_loop(0, _CHUNK, zrow, 0)

    def ztmp(t, _):
        degb[pl.ds(t * _L, _L)] = zv
        return 0
    lax.fori_loop(0, _RPT // _L, ztmp, 0)

    def orow(t, _):
        onesb[pl.ds(t * _L, _L)] = jnp.ones((_L,), jnp.float32)
        return 0
    lax.fori_loop(0, _CHUNK // _L, orow, 0)

    pltpu.sync_copy(degb.at[pl.ds(0, _RPT)], deg_s.at[pl.ds(r0, _RPT)])

    def zs(c, _):
        pltpu.sync_copy(zbuf, s_s.at[pl.ds(r0 + c * _CHUNK, _CHUNK)])
        return 0
    lax.fori_loop(0, _CPB, zs, 0)
    plsc.subcore_barrier()

    def degch(j, _):
        pltpu.async_copy(onesb, deg_s.at[coli.at[j]], ssem, add=True)
        return 0
    lax.fori_loop(0, _CH, degch, 0)

    def degdr(j, _):
        pltpu.make_async_copy(onesb, deg_s.at[coli.at[0]], ssem).wait()
        return 0
    lax.fori_loop(0, _CH, degdr, 0)
    plsc.subcore_barrier()

    pltpu.sync_copy(deg_s.at[pl.ds(r0, _RPT)], degb.at[pl.ds(0, _RPT)])

    def disrow(t, _):
        d = degb[pl.ds(t * _L, _L)] + jnp.float32(1.0)
        y = _nrsqrt(d)
        degb[pl.ds(t * _L, _L)] = y
        d2b[pl.ds(t * _L, _L)] = jnp.float32(0.9) / d
        idb[pl.ds(t * _L, _L)] = d * y
        return 0
    lax.fori_loop(0, _RPT // _L, disrow, 0)

    def initch(c, _):
        pltpu.sync_copy(
            h0s.at[cid, pl.ds(r0 + c * _CHUNK, _CHUNK)], buf.at[0])

        def initrow(i, _):
            r = c * _CHUNK + i
            dis = _sload(degb, r)
            v0 = buf[0, i, pl.ds(0, _L)] * dis
            v1 = buf[0, i, pl.ds(_L, _L)] * dis
            ga[r, pl.ds(0, _L)] = v0
            ga[r, pl.ds(_L, _L)] = v1
            g0p[r, pl.ds(0, _L)] = v0 * jnp.float32(0.1)
            g0p[r, pl.ds(_L, _L)] = v1 * jnp.float32(0.1)
            return 0
        lax.fori_loop(0, _CHUNK, initrow, 0)
        return 0
    lax.fori_loop(0, _CPB, initch, 0)
    pltpu.sync_copy(ga, g_s.at[pl.ds(r0, _RPT)])
    plsc.subcore_barrier()

    def kstep(k, _):
        pltpu.async_copy(g_s.at[rowi.at[0]], buf.at[0], gsem)

        def ech(j, _):
            @pl.when(j > 0)
            def _():
                pltpu.make_async_copy(
                    buf.at[0], s_s.at[coli.at[0]], ssem).wait()

            @pl.when(j + 1 < _CH)
            def _():
                pltpu.async_copy(
                    g_s.at[rowi.at[j + 1]], buf.at[(j + 1) % 2], gsem)
            pltpu.make_async_copy(
                g_s.at[rowi.at[0]], buf.at[0], gsem).wait()
            pltpu.async_copy(
                buf.at[j % 2], s_s.at[coli.at[j]], ssem, add=True)
            return 0
        lax.fori_loop(0, _CH, ech, 0)
        pltpu.make_async_copy(buf.at[0], s_s.at[coli.at[0]], ssem).wait()
        plsc.subcore_barrier()

        def dchunk(c, _):
            base = c * _CHUNK
            pltpu.sync_copy(s_s.at[pl.ds(r0 + base, _CHUNK)], buf.at[0])
            pltpu.sync_copy(zbuf, s_s.at[pl.ds(r0 + base, _CHUNK)])

            def drow(i, _):
                r = base + i
                d2 = _sload(d2b, r)
                v0 = (buf[0, i, pl.ds(0, _L)] + ga[r, pl.ds(0, _L)]) * d2 \
                    + g0p[r, pl.ds(0, _L)]
                v1 = (buf[0, i, pl.ds(_L, _L)] + ga[r, pl.ds(_L, _L)]) * d2 \
                    + g0p[r, pl.ds(_L, _L)]
                ga[r, pl.ds(0, _L)] = v0
                ga[r, pl.ds(_L, _L)] = v1
                return 0
            lax.fori_loop(0, _CHUNK, drow, 0)
            return 0
        lax.fori_loop(0, _CPB, dchunk, 0)
        pltpu.sync_copy(ga, g_s.at[pl.ds(r0, _RPT)])
        plsc.subcore_barrier()
        return 0
    lax.fori_loop(0, _K, kstep, 0)

    def bnrow(i, carry):
        s0, s1, q0, q1 = carry
        inv = _sload(idb, i)
        h0v = ga[i, pl.ds(0, _L)] * inv
        h1v = ga[i, pl.ds(_L, _L)] * inv
        return (s0 + h0v, s1 + h1v, q0 + h0v * h0v, q1 + h1v * h1v)
    s0, s1, q0, q1 = lax.fori_loop(0, _RPT, bnrow, (zv, zv, zv, zv))
    bnw[pl.ds(0, _L)] = s0
    bnw[pl.ds(_L, _L)] = s1
    bnw[pl.ds(2 * _L, _L)] = q0
    bnw[pl.ds(3 * _L, _L)] = q1
    pltpu.sync_copy(bnw, bn_s.at[sid])
    plsc.subcore_barrier()
    pltpu.sync_copy(bn_s, bnl)

    def red(t, carry):
        s0, s1, q0, q1 = carry
        return (s0 + bnl[t, pl.ds(0, _L)],
                s1 + bnl[t, pl.ds(_L, _L)],
                q0 + bnl[t, pl.ds(2 * _L, _L)],
                q1 + bnl[t, pl.ds(3 * _L, _L)])
    s0, s1, q0, q1 = lax.fori_loop(0, _NS, red, (zv, zv, zv, zv))
    ninv = jnp.float32(1.0 / _N)
    m0 = s0 * ninv
    m1 = s1 * ninv
    v0 = q0 * ninv - m0 * m0
    v1 = q1 * ninv - m1 * m1
    sc0 = gbl[0, pl.ds(0, _L)] * _nrsqrt(v0 + jnp.float32(_EPS))
    sc1 = gbl[0, pl.ds(_L, _L)] * _nrsqrt(v1 + jnp.float32(_EPS))
    sh0 = gbl[1, pl.ds(0, _L)] - sc0 * m0
    sh1 = gbl[1, pl.ds(_L, _L)] - sc1 * m1

    def arow(i, _):
        inv = _sload(idb, i)
        ga[i, pl.ds(0, _L)] = (ga[i, pl.ds(0, _L)] * inv) * sc0 + sh0
        ga[i, pl.ds(_L, _L)] = (ga[i, pl.ds(_L, _L)] * inv) * sc1 + sh1
        return 0
    lax.fori_loop(0, _RPT, arow, 0)
    pltpu.sync_copy(ga, out_h.at[cid, pl.ds(r0, _RPT)])


def kernel(x, edge_index, W, b, gamma, beta):
    f32 = jnp.float32
    x_pad = jnp.pad(x, ((0, _NPAD - _N), (0, 0)))
    b2 = b.reshape(_NC, 1, _DH)

    h0s = pl.pallas_call(
        _mm_body,
        grid=(_NPAD // _MMBLK, _NC),
        in_specs=[
            pl.BlockSpec((_MMBLK, _D_IN), lambda i, c: (i, 0)),
            pl.BlockSpec((_DH, _D_IN), lambda i, c: (c, 0)),
            pl.BlockSpec((1, 1, _DH), lambda i, c: (c, 0, 0)),
        ],
        out_specs=pl.BlockSpec((1, _MMBLK, _DH), lambda i, c: (c, i, 0)),
        out_shape=jax.ShapeDtypeStruct((_NC, _NPAD, _DH), f32),
    )(x_pad, W, b2)

    padn = _EPAD - _E
    pr = _N + (jnp.arange(padn, dtype=jnp.int32) % 112)
    pc = _N + 112 + (jnp.arange(padn, dtype=jnp.int32) % 112)
    rows_p = jnp.concatenate([edge_index[0], pr]).reshape(_NS, _CH, _CHUNK)
    cols_p = jnp.concatenate([edge_index[1], pc]).reshape(_NS, _CH, _CHUNK)
    gb = jnp.stack([gamma.reshape(_NC, _DH), beta.reshape(_NC, _DH)], axis=1)

    mesh = plsc.VectorSubcoreMesh(core_axis_name="c", subcore_axis_name="s")
    out = pl.kernel(
        _sc_body,
        out_type=jax.ShapeDtypeStruct((_NC, _NPAD, _DH), f32),
        mesh=mesh,
        compiler_params=pltpu.CompilerParams(use_tc_tiling_on_sc=False),
        scratch_types=[
            pltpu.VMEM_SHARED((_NPAD, _DH), f32),
            pltpu.VMEM_SHARED((_NPAD, _DH), f32),
            pltpu.VMEM_SHARED((_NPAD,), f32),
            pltpu.VMEM_SHARED((_NS, 4 * _L), f32),
            pltpu.VMEM((_CH, _CHUNK), jnp.int32),
            pltpu.VMEM((_CH, _CHUNK), jnp.int32),
            pltpu.VMEM((2, _CHUNK, _DH), f32),
            pltpu.VMEM((_CHUNK, _DH), f32),
            pltpu.VMEM((_RPT, _DH), f32),
            pltpu.VMEM((_RPT, _DH), f32),
            pltpu.VMEM((_RPT + _L,), f32),
            pltpu.VMEM((_RPT + _L,), f32),
            pltpu.VMEM((_RPT + _L,), f32),
            pltpu.VMEM((_CHUNK,), f32),
            pltpu.VMEM((4 * _L,), f32),
            pltpu.VMEM((_NS, 4 * _L), f32),
            pltpu.VMEM((2, _DH), f32),
            pltpu.SemaphoreType.DMA,
            pltpu.SemaphoreType.DMA,
        ],
    )(h0s, rows_p, cols_p, gb)

    return jnp.concatenate([out[0, :_N], out[1, :_N]], axis=1)


# speedup vs baseline: 31.5588x; 1.1144x over previous
import jax
import jax.numpy as jnp
from jax import lax
from jax.experimental import pallas as pl
from jax.experimental.pallas import tpu as pltpu
from jax.experimental.pallas import tpu_sc as plsc

_N = 10000
_E = 160000
_D_IN = 256
_D_OUT = 64
_K = 10
_EPS = 1e-5

_NC = 2
_NS = 16
_L = 16
_NPAD = 10240
_RPT = _NPAD // _NS
_DH = _D_OUT // _NC
_CHUNK = 128
_CH = 79
_EPT = _CH * _CHUNK
_EPAD = _EPT * _NS

_MMBLK = 2048


def _mm_body(x_ref, w_ref, b_ref, o_ref):
    i = pl.program_id(0)
    acc = lax.dot_general(
        x_ref[...], w_ref[...], (((1,), (1,)), ((), ())),
        preferred_element_type=jnp.float32)
    rowid = i * _MMBLK + lax.broadcasted_iota(jnp.int32, (_MMBLK, _DH), 0)
    o_ref[...] = jnp.where(rowid < _N, acc + b_ref[0], 0.0)[None]


def _nrsqrt(x):
    i = lax.bitcast_convert_type(x, jnp.int32)
    i = jnp.int32(0x5F3759DF) - (i >> 1)
    y = lax.bitcast_convert_type(i, jnp.float32)
    for _ in range(3):
        y = y * (jnp.float32(1.5) - jnp.float32(0.5) * x * y * y)
    return y


def _sload(ref, i):
    return ref[pl.ds(i, _L)][0]


_CPB = _RPT // _CHUNK
_ZR = 64


def _sc_body(h0s, rows_h, cols_h, gb_h, out_h,
             g_s, s_s, deg_s, bn_s,
             rowi, coli, buf, zbuf, ga, g0p, degb, d2b, idb, onesb,
             bnw, bnl, gbl, gsem, ssem):
    cid = lax.axis_index("c")
    sid = lax.axis_index("s")
    r0 = sid * _RPT

    pltpu.sync_copy(rows_h.at[sid], rowi)
    pltpu.sync_copy(cols_h.at[sid], coli)
    pltpu.sync_copy(gb_h.at[cid], gbl)

    zv = jnp.zeros((_L,), jnp.float32)

    def zrow(i, _):
        zbuf[i, pl.ds(0, _L)] = zv
        zbuf[i, pl.ds(_L, _L)] = zv
        return 0
    lax.fori_loop(0, _ZR, zrow, 0)

    def ztmp(t, _):
        degb[pl.ds(t * _L, _L)] = zv
        return 0
    lax.fori_loop(0, _RPT // _L, ztmp, 0)

    def orow(t, _):
        onesb[pl.ds(t * _L, _L)] = jnp.ones((_L,), jnp.float32)
        return 0
    lax.fori_loop(0, _CHUNK // _L, orow, 0)

    pltpu.sync_copy(degb.at[pl.ds(0, _RPT)], deg_s.at[pl.ds(r0, _RPT)])

    def zs(c, _):
        pltpu.sync_copy(zbuf, s_s.at[pl.ds(r0 + c * _ZR, _ZR)])
        return 0
    lax.fori_loop(0, _RPT // _ZR, zs, 0)
    plsc.subcore_barrier()

    def degch(j, _):
        pltpu.async_copy(onesb, deg_s.at[coli.at[j]], ssem, add=True)
        return 0
    lax.fori_loop(0, _CH, degch, 0)

    def degdr(j, _):
        pltpu.make_async_copy(onesb, deg_s.at[coli.at[0]], ssem).wait()
        return 0
    lax.fori_loop(0, _CH, degdr, 0)
    plsc.subcore_barrier()

    pltpu.sync_copy(deg_s.at[pl.ds(r0, _RPT)], degb.at[pl.ds(0, _RPT)])

    def disrow(t, _):
        d = degb[pl.ds(t * _L, _L)] + jnp.float32(1.0)
        y = _nrsqrt(d)
        degb[pl.ds(t * _L, _L)] = y
        d2b[pl.ds(t * _L, _L)] = jnp.float32(0.9) / d
        idb[pl.ds(t * _L, _L)] = d * y
        return 0
    lax.fori_loop(0, _RPT // _L, disrow, 0)

    def initch(c, _):
        pltpu.sync_copy(
            h0s.at[cid, pl.ds(r0 + c * _CHUNK, _CHUNK)], buf.at[0])

        def initrow(i, _):
            r = c * _CHUNK + i
            dis = _sload(degb, r)
            v0 = buf[0, i, pl.ds(0, _L)] * dis
            v1 = buf[0, i, pl.ds(_L, _L)] * dis
            ga[r, pl.ds(0, _L)] = v0
            ga[r, pl.ds(_L, _L)] = v1
            g0p[r, pl.ds(0, _L)] = v0 * jnp.float32(0.1)
            g0p[r, pl.ds(_L, _L)] = v1 * jnp.float32(0.1)
            return 0
        lax.fori_loop(0, _CHUNK, initrow, 0)
        return 0
    lax.fori_loop(0, _CPB, initch, 0)
    pltpu.sync_copy(ga, g_s.at[pl.ds(r0, _RPT)])
    plsc.subcore_barrier()

    def kstep(k, _):
        pltpu.async_copy(g_s.at[rowi.at[0]], buf.at[0], gsem)
        pltpu.async_copy(g_s.at[rowi.at[1]], buf.at[1], gsem)

        def ech(j, _):
            pltpu.make_async_copy(
                g_s.at[rowi.at[0]], buf.at[0], gsem).wait()
            pltpu.async_copy(
                buf.at[j % 3], s_s.at[coli.at[j]], ssem, add=True)

            @pl.when(j > 0)
            def _():
                pltpu.make_async_copy(
                    buf.at[0], s_s.at[coli.at[0]], ssem).wait()

            @pl.when(j + 2 < _CH)
            def _():
                pltpu.async_copy(
                    g_s.at[rowi.at[j + 2]], buf.at[(j + 2) % 3], gsem)
            return 0
        lax.fori_loop(0, _CH, ech, 0)
        pltpu.make_async_copy(buf.at[0], s_s.at[coli.at[0]], ssem).wait()
        plsc.subcore_barrier()

        def dchunk(c, _):
            base = c * _CHUNK
            pltpu.sync_copy(s_s.at[pl.ds(r0 + base, _CHUNK)], buf.at[0])
            pltpu.sync_copy(zbuf, s_s.at[pl.ds(r0 + base, _ZR)])
            pltpu.sync_copy(zbuf, s_s.at[pl.ds(r0 + base + _ZR, _ZR)])

            def drow(i, _):
                r = base + i
                d2 = _sload(d2b, r)
                v0 = (buf[0, i, pl.ds(0, _L)] + ga[r, pl.ds(0, _L)]) * d2 \
                    + g0p[r, pl.ds(0, _L)]
                v1 = (buf[0, i, pl.ds(_L, _L)] + ga[r, pl.ds(_L, _L)]) * d2 \
                    + g0p[r, pl.ds(_L, _L)]
                ga[r, pl.ds(0, _L)] = v0
                ga[r, pl.ds(_L, _L)] = v1
                return 0
            lax.fori_loop(0, _CHUNK, drow, 0)
            return 0
        lax.fori_loop(0, _CPB, dchunk, 0)
        pltpu.sync_copy(ga, g_s.at[pl.ds(r0, _RPT)])
        plsc.subcore_barrier()
        return 0
    lax.fori_loop(0, _K, kstep, 0)

    def bnrow(i, carry):
        s0, s1, q0, q1 = carry
        inv = _sload(idb, i)
        h0v = ga[i, pl.ds(0, _L)] * inv
        h1v = ga[i, pl.ds(_L, _L)] * inv
        return (s0 + h0v, s1 + h1v, q0 + h0v * h0v, q1 + h1v * h1v)
    s0, s1, q0, q1 = lax.fori_loop(0, _RPT, bnrow, (zv, zv, zv, zv))
    bnw[pl.ds(0, _L)] = s0
    bnw[pl.ds(_L, _L)] = s1
    bnw[pl.ds(2 * _L, _L)] = q0
    bnw[pl.ds(3 * _L, _L)] = q1
    pltpu.sync_copy(bnw, bn_s.at[sid])
    plsc.subcore_barrier()
    pltpu.sync_copy(bn_s, bnl)

    def red(t, carry):
        s0, s1, q0, q1 = carry
        return (s0 + bnl[t, pl.ds(0, _L)],
                s1 + bnl[t, pl.ds(_L, _L)],
                q0 + bnl[t, pl.ds(2 * _L, _L)],
                q1 + bnl[t, pl.ds(3 * _L, _L)])
    s0, s1, q0, q1 = lax.fori_loop(0, _NS, red, (zv, zv, zv, zv))
    ninv = jnp.float32(1.0 / _N)
    m0 = s0 * ninv
    m1 = s1 * ninv
    v0 = q0 * ninv - m0 * m0
    v1 = q1 * ninv - m1 * m1
    sc0 = gbl[0, pl.ds(0, _L)] * _nrsqrt(v0 + jnp.float32(_EPS))
    sc1 = gbl[0, pl.ds(_L, _L)] * _nrsqrt(v1 + jnp.float32(_EPS))
    sh0 = gbl[1, pl.ds(0, _L)] - sc0 * m0
    sh1 = gbl[1, pl.ds(_L, _L)] - sc1 * m1

    def arow(i, _):
        inv = _sload(idb, i)
        ga[i, pl.ds(0, _L)] = (ga[i, pl.ds(0, _L)] * inv) * sc0 + sh0
        ga[i, pl.ds(_L, _L)] = (ga[i, pl.ds(_L, _L)] * inv) * sc1 + sh1
        return 0
    lax.fori_loop(0, _RPT, arow, 0)
    pltpu.sync_copy(ga, out_h.at[cid, pl.ds(r0, _RPT)])


def kernel(x, edge_index, W, b, gamma, beta):
    f32 = jnp.float32
    x_pad = jnp.pad(x, ((0, _NPAD - _N), (0, 0)))
    b2 = b.reshape(_NC, 1, _DH)

    h0s = pl.pallas_call(
        _mm_body,
        grid=(_NPAD // _MMBLK, _NC),
        in_specs=[
            pl.BlockSpec((_MMBLK, _D_IN), lambda i, c: (i, 0)),
            pl.BlockSpec((_DH, _D_IN), lambda i, c: (c, 0)),
            pl.BlockSpec((1, 1, _DH), lambda i, c: (c, 0, 0)),
        ],
        out_specs=pl.BlockSpec((1, _MMBLK, _DH), lambda i, c: (c, i, 0)),
        out_shape=jax.ShapeDtypeStruct((_NC, _NPAD, _DH), f32),
    )(x_pad, W, b2)

    padn = _EPAD - _E
    pr = _N + (jnp.arange(padn, dtype=jnp.int32) % 112)
    pc = _N + 112 + (jnp.arange(padn, dtype=jnp.int32) % 112)
    rows_p = jnp.concatenate([edge_index[0], pr]).reshape(_NS, _CH, _CHUNK)
    cols_p = jnp.concatenate([edge_index[1], pc]).reshape(_NS, _CH, _CHUNK)
    gb = jnp.stack([gamma.reshape(_NC, _DH), beta.reshape(_NC, _DH)], axis=1)

    mesh = plsc.VectorSubcoreMesh(core_axis_name="c", subcore_axis_name="s")
    out = pl.kernel(
        _sc_body,
        out_type=jax.ShapeDtypeStruct((_NC, _NPAD, _DH), f32),
        mesh=mesh,
        compiler_params=pltpu.CompilerParams(use_tc_tiling_on_sc=False),
        scratch_types=[
            pltpu.VMEM_SHARED((_NPAD, _DH), f32),
            pltpu.VMEM_SHARED((_NPAD, _DH), f32),
            pltpu.VMEM_SHARED((_NPAD,), f32),
            pltpu.VMEM_SHARED((_NS, 4 * _L), f32),
            pltpu.VMEM((_CH, _CHUNK), jnp.int32),
            pltpu.VMEM((_CH, _CHUNK), jnp.int32),
            pltpu.VMEM((3, _CHUNK, _DH), f32),
            pltpu.VMEM((_ZR, _DH), f32),
            pltpu.VMEM((_RPT, _DH), f32),
            pltpu.VMEM((_RPT, _DH), f32),
            pltpu.VMEM((_RPT + _L,), f32),
            pltpu.VMEM((_RPT + _L,), f32),
            pltpu.VMEM((_RPT + _L,), f32),
            pltpu.VMEM((_CHUNK,), f32),
            pltpu.VMEM((4 * _L,), f32),
            pltpu.VMEM((_NS, 4 * _L), f32),
            pltpu.VMEM((2, _DH), f32),
            pltpu.SemaphoreType.DMA,
            pltpu.SemaphoreType.DMA,
        ],
    )(h0s, rows_p, cols_p, gb)

    return jnp.concatenate([out[0, :_N], out[1, :_N]], axis=1)


# speedup vs baseline: 34.5060x; 1.0934x over previous
import jax
import jax.numpy as jnp
from jax import lax
from jax.experimental import pallas as pl
from jax.experimental.pallas import tpu as pltpu
from jax.experimental.pallas import tpu_sc as plsc

_N = 10000
_E = 160000
_D_IN = 256
_D_OUT = 64
_K = 10
_EPS = 1e-5

_NC = 2
_NS = 16
_L = 16
_NPAD = 10240
_RPT = _NPAD // _NS
_DH = _D_OUT // _NC
_CHUNK = 128
_CH = 79
_EPT = _CH * _CHUNK
_EPAD = _EPT * _NS

_MMBLK = 2048


def _mm_body(x_ref, w_ref, b_ref, o_ref):
    i = pl.program_id(0)
    acc = lax.dot_general(
        x_ref[...], w_ref[...], (((1,), (1,)), ((), ())),
        preferred_element_type=jnp.float32)
    rowid = i * _MMBLK + lax.broadcasted_iota(jnp.int32, (_MMBLK, _DH), 0)
    o_ref[...] = jnp.where(rowid < _N, acc + b_ref[0], 0.0)[None]


def _nrsqrt(x):
    i = lax.bitcast_convert_type(x, jnp.int32)
    i = jnp.int32(0x5F3759DF) - (i >> 1)
    y = lax.bitcast_convert_type(i, jnp.float32)
    for _ in range(3):
        y = y * (jnp.float32(1.5) - jnp.float32(0.5) * x * y * y)
    return y


def _sload(ref, i):
    return ref[pl.ds(i, _L)][0]


_CPB = _RPT // _CHUNK
_ZR = 64
_LAST = _N - (_NS - 1) * _RPT


def _sc_body(h0s, rows_h, cols_h, gb_h, out_h,
             g_s, s_s, deg_s, bn_s,
             rowi, coli, buf, zbuf, ga, g0p, degb, d2b, idb, onesb,
             bnw, bnl, gbl, gsem, ssem):
    cid = lax.axis_index("c")
    sid = lax.axis_index("s")
    r0 = sid * _RPT

    pltpu.sync_copy(rows_h.at[sid], rowi)
    pltpu.sync_copy(cols_h.at[sid], coli)
    pltpu.sync_copy(gb_h.at[cid], gbl)

    zv = jnp.zeros((_L,), jnp.float32)

    def zrow(i, _):
        zbuf[i, pl.ds(0, _L)] = zv
        zbuf[i, pl.ds(_L, _L)] = zv
        return 0
    lax.fori_loop(0, _ZR, zrow, 0)

    def ztmp(t, _):
        degb[pl.ds(t * _L, _L)] = zv
        return 0
    lax.fori_loop(0, _RPT // _L, ztmp, 0)

    def orow(t, _):
        onesb[pl.ds(t * _L, _L)] = jnp.ones((_L,), jnp.float32)
        return 0
    lax.fori_loop(0, _CHUNK // _L, orow, 0)

    pltpu.sync_copy(degb.at[pl.ds(0, _RPT)], deg_s.at[pl.ds(r0, _RPT)])

    def zs(c, _):
        pltpu.sync_copy(zbuf, s_s.at[pl.ds(r0 + c * _ZR, _ZR)])
        return 0
    lax.fori_loop(0, _RPT // _ZR, zs, 0)
    plsc.subcore_barrier()

    def degch(j, _):
        pltpu.async_copy(onesb, deg_s.at[coli.at[j]], ssem, add=True)
        return 0
    lax.fori_loop(0, _CH, degch, 0)

    def degdr(j, _):
        pltpu.make_async_copy(onesb, deg_s.at[coli.at[0]], ssem).wait()
        return 0
    lax.fori_loop(0, _CH, degdr, 0)
    plsc.subcore_barrier()

    pltpu.sync_copy(deg_s.at[pl.ds(r0, _RPT)], degb.at[pl.ds(0, _RPT)])

    def disrow(t, _):
        d = degb[pl.ds(t * _L, _L)] + jnp.float32(1.0)
        y = _nrsqrt(d)
        degb[pl.ds(t * _L, _L)] = y
        d2b[pl.ds(t * _L, _L)] = jnp.float32(0.9) / d
        idb[pl.ds(t * _L, _L)] = d * y
        return 0
    lax.fori_loop(0, _RPT // _L, disrow, 0)

    def initch(c, _):
        pltpu.sync_copy(
            h0s.at[cid, pl.ds(r0 + c * _CHUNK, _CHUNK)], buf.at[0])

        def initrow(i, _):
            r = c * _CHUNK + i
            dis = _sload(degb, r)
            v0 = buf[0, i, pl.ds(0, _L)] * dis
            v1 = buf[0, i, pl.ds(_L, _L)] * dis
            ga[r, pl.ds(0, _L)] = v0
            ga[r, pl.ds(_L, _L)] = v1
            g0p[r, pl.ds(0, _L)] = v0 * jnp.float32(0.1)
            g0p[r, pl.ds(_L, _L)] = v1 * jnp.float32(0.1)
            return 0
        lax.fori_loop(0, _CHUNK, initrow, 0)
        return 0
    lax.fori_loop(0, _CPB, initch, 0)
    pltpu.sync_copy(ga, g_s.at[pl.ds(r0, _RPT)])
    plsc.subcore_barrier()

    def kstep(k, _):
        pltpu.async_copy(g_s.at[rowi.at[0]], buf.at[0], gsem)
        pltpu.async_copy(g_s.at[rowi.at[1]], buf.at[1], gsem)

        def ech(j, _):
            pltpu.make_async_copy(
                g_s.at[rowi.at[0]], buf.at[0], gsem).wait()
            pltpu.async_copy(
                buf.at[j % 3], s_s.at[coli.at[j]], ssem, add=True)

            @pl.when(j > 0)
            def _():
                pltpu.make_async_copy(
                    buf.at[0], s_s.at[coli.at[0]], ssem).wait()

            @pl.when(j + 2 < _CH)
            def _():
                pltpu.async_copy(
                    g_s.at[rowi.at[j + 2]], buf.at[(j + 2) % 3], gsem)
            return 0
        lax.fori_loop(0, _CH, ech, 0)
        pltpu.make_async_copy(buf.at[0], s_s.at[coli.at[0]], ssem).wait()
        plsc.subcore_barrier()

        pltpu.async_copy(s_s.at[pl.ds(r0, _CHUNK)], buf.at[0], gsem)

        def dchunk(c, _):
            base = c * _CHUNK
            pltpu.make_async_copy(
                s_s.at[pl.ds(r0, _CHUNK)], buf.at[0], gsem).wait()

            @pl.when(c + 1 < _CPB)
            def _():
                pltpu.async_copy(
                    s_s.at[pl.ds(r0 + base + _CHUNK, _CHUNK)],
                    buf.at[(c + 1) % 3], gsem)
            pltpu.async_copy(zbuf, s_s.at[pl.ds(r0 + base, _ZR)], ssem)
            pltpu.async_copy(
                zbuf, s_s.at[pl.ds(r0 + base + _ZR, _ZR)], ssem)

            def drow(i, _):
                r = base + i
                d2 = _sload(d2b, r)
                v0 = (buf[c % 3, i, pl.ds(0, _L)] + ga[r, pl.ds(0, _L)]) \
                    * d2 + g0p[r, pl.ds(0, _L)]
                v1 = (buf[c % 3, i, pl.ds(_L, _L)] + ga[r, pl.ds(_L, _L)]) \
                    * d2 + g0p[r, pl.ds(_L, _L)]
                ga[r, pl.ds(0, _L)] = v0
                ga[r, pl.ds(_L, _L)] = v1
                return 0
            lax.fori_loop(0, _CHUNK, drow, 0)
            pltpu.async_copy(
                ga.at[pl.ds(base, _CHUNK)],
                g_s.at[pl.ds(r0 + base, _CHUNK)], ssem)
            return 0
        lax.fori_loop(0, _CPB, dchunk, 0)

        def ddr(c, _):
            pltpu.make_async_copy(zbuf, s_s.at[pl.ds(r0, _ZR)], ssem).wait()
            pltpu.make_async_copy(zbuf, s_s.at[pl.ds(r0, _ZR)], ssem).wait()
            pltpu.make_async_copy(
                ga.at[pl.ds(0, _CHUNK)],
                g_s.at[pl.ds(r0, _CHUNK)], ssem).wait()
            return 0
        lax.fori_loop(0, _CPB, ddr, 0)
        plsc.subcore_barrier()
        return 0
    lax.fori_loop(0, _K, kstep, 0)

    def bnrow(i, carry):
        s0, s1, q0, q1 = carry
        inv = _sload(idb, i)
        h0v = ga[i, pl.ds(0, _L)] * inv
        h1v = ga[i, pl.ds(_L, _L)] * inv
        return (s0 + h0v, s1 + h1v, q0 + h0v * h0v, q1 + h1v * h1v)
    s0, s1, q0, q1 = lax.fori_loop(0, _RPT, bnrow, (zv, zv, zv, zv))
    bnw[pl.ds(0, _L)] = s0
    bnw[pl.ds(_L, _L)] = s1
    bnw[pl.ds(2 * _L, _L)] = q0
    bnw[pl.ds(3 * _L, _L)] = q1
    pltpu.sync_copy(bnw, bn_s.at[sid])
    plsc.subcore_barrier()
    pltpu.sync_copy(bn_s, bnl)

    def red(t, carry):
        s0, s1, q0, q1 = carry
        return (s0 + bnl[t, pl.ds(0, _L)],
                s1 + bnl[t, pl.ds(_L, _L)],
                q0 + bnl[t, pl.ds(2 * _L, _L)],
                q1 + bnl[t, pl.ds(3 * _L, _L)])
    s0, s1, q0, q1 = lax.fori_loop(0, _NS, red, (zv, zv, zv, zv))
    ninv = jnp.float32(1.0 / _N)
    m0 = s0 * ninv
    m1 = s1 * ninv
    v0 = q0 * ninv - m0 * m0
    v1 = q1 * ninv - m1 * m1
    sc0 = gbl[0, pl.ds(0, _L)] * _nrsqrt(v0 + jnp.float32(_EPS))
    sc1 = gbl[0, pl.ds(_L, _L)] * _nrsqrt(v1 + jnp.float32(_EPS))
    sh0 = gbl[1, pl.ds(0, _L)] - sc0 * m0
    sh1 = gbl[1, pl.ds(_L, _L)] - sc1 * m1

    def arow(i, _):
        inv = _sload(idb, i)
        ga[i, pl.ds(0, _L)] = (ga[i, pl.ds(0, _L)] * inv) * sc0 + sh0
        ga[i, pl.ds(_L, _L)] = (ga[i, pl.ds(_L, _L)] * inv) * sc1 + sh1
        return 0
    lax.fori_loop(0, _RPT, arow, 0)
    c0 = cid * _DH

    @pl.when(sid < _NS - 1)
    def _():
        pltpu.sync_copy(
            ga.at[pl.ds(0, _RPT)],
            out_h.at[pl.ds(r0, _RPT), pl.ds(c0, _DH)])

    @pl.when(sid == _NS - 1)
    def _():
        pltpu.sync_copy(
            ga.at[pl.ds(0, _LAST)],
            out_h.at[pl.ds(r0, _LAST), pl.ds(c0, _DH)])


def kernel(x, edge_index, W, b, gamma, beta):
    f32 = jnp.float32
    x_pad = jnp.pad(x, ((0, _NPAD - _N), (0, 0)))
    b2 = b.reshape(_NC, 1, _DH)

    h0s = pl.pallas_call(
        _mm_body,
        grid=(_NPAD // _MMBLK, _NC),
        in_specs=[
            pl.BlockSpec((_MMBLK, _D_IN), lambda i, c: (i, 0)),
            pl.BlockSpec((_DH, _D_IN), lambda i, c: (c, 0)),
            pl.BlockSpec((1, 1, _DH), lambda i, c: (c, 0, 0)),
        ],
        out_specs=pl.BlockSpec((1, _MMBLK, _DH), lambda i, c: (c, i, 0)),
        out_shape=jax.ShapeDtypeStruct((_NC, _NPAD, _DH), f32),
    )(x_pad, W, b2)

    padn = _EPAD - _E
    pr = _N + (jnp.arange(padn, dtype=jnp.int32) % 112)
    pc = _N + 112 + (jnp.arange(padn, dtype=jnp.int32) % 112)
    rows_p = jnp.concatenate([edge_index[0], pr]).reshape(_NS, _CH, _CHUNK)
    cols_p = jnp.concatenate([edge_index[1], pc]).reshape(_NS, _CH, _CHUNK)
    gb = jnp.stack([gamma.reshape(_NC, _DH), beta.reshape(_NC, _DH)], axis=1)

    mesh = plsc.VectorSubcoreMesh(core_axis_name="c", subcore_axis_name="s")
    out = pl.kernel(
        _sc_body,
        out_type=jax.ShapeDtypeStruct((_N, _D_OUT), f32),
        mesh=mesh,
        compiler_params=pltpu.CompilerParams(use_tc_tiling_on_sc=False),
        scratch_types=[
            pltpu.VMEM_SHARED((_NPAD, _DH), f32),
            pltpu.VMEM_SHARED((_NPAD, _DH), f32),
            pltpu.VMEM_SHARED((_NPAD,), f32),
            pltpu.VMEM_SHARED((_NS, 4 * _L), f32),
            pltpu.VMEM((_CH, _CHUNK), jnp.int32),
            pltpu.VMEM((_CH, _CHUNK), jnp.int32),
            pltpu.VMEM((3, _CHUNK, _DH), f32),
            pltpu.VMEM((_ZR, _DH), f32),
            pltpu.VMEM((_RPT, _DH), f32),
            pltpu.VMEM((_RPT, _DH), f32),
            pltpu.VMEM((_RPT + _L,), f32),
            pltpu.VMEM((_RPT + _L,), f32),
            pltpu.VMEM((_RPT + _L,), f32),
            pltpu.VMEM((_CHUNK,), f32),
            pltpu.VMEM((4 * _L,), f32),
            pltpu.VMEM((_NS, 4 * _L), f32),
            pltpu.VMEM((2, _DH), f32),
            pltpu.SemaphoreType.DMA,
            pltpu.SemaphoreType.DMA,
        ],
    )(h0s, rows_p, cols_p, gb)

    return out


# speedup vs baseline: 34.9133x; 1.0118x over previous
import jax
import jax.numpy as jnp
from jax import lax
from jax.experimental import pallas as pl
from jax.experimental.pallas import tpu as pltpu
from jax.experimental.pallas import tpu_sc as plsc

_N = 10000
_E = 160000
_D_IN = 256
_D_OUT = 64
_K = 10
_EPS = 1e-5

_NC = 2
_NS = 16
_L = 16
_NPAD = 10240
_RPT = _NPAD // _NS
_DH = _D_OUT // _NC
_CHUNK = 128
_CH = 79
_EPT = _CH * _CHUNK
_EPAD = _EPT * _NS

_MMBLK = 2048


def _mm_body(x_ref, w_ref, b_ref, o_ref):
    i = pl.program_id(0)
    acc = lax.dot_general(
        x_ref[...], w_ref[...], (((1,), (1,)), ((), ())),
        preferred_element_type=jnp.float32)
    rowid = i * _MMBLK + lax.broadcasted_iota(jnp.int32, (_MMBLK, _DH), 0)
    o_ref[...] = jnp.where(rowid < _N, acc + b_ref[0], 0.0)[None]


def _nrsqrt(x):
    i = lax.bitcast_convert_type(x, jnp.int32)
    i = jnp.int32(0x5F3759DF) - (i >> 1)
    y = lax.bitcast_convert_type(i, jnp.float32)
    for _ in range(3):
        y = y * (jnp.float32(1.5) - jnp.float32(0.5) * x * y * y)
    return y


def _sload(ref, i):
    return ref[pl.ds(i, _L)][0]


_CPB = _RPT // _CHUNK
_ZR = 64
_LAST = _N - (_NS - 1) * _RPT


def _sc_body(h0s, rows_h, cols_h, gb_h, out_h,
             g_s, s_s, deg_s, bn_s,
             rowi, coli, buf, zbuf, ga, g0p, degb, d2b, idb, onesb,
             bnw, bnl, gbl, gsem, ssem):
    cid = lax.axis_index("c")
    sid = lax.axis_index("s")
    r0 = sid * _RPT

    pltpu.sync_copy(rows_h.at[sid], rowi)
    pltpu.sync_copy(cols_h.at[sid], coli)
    pltpu.sync_copy(gb_h.at[cid], gbl)

    zv = jnp.zeros((_L,), jnp.float32)

    def zrow(i, _):
        zbuf[i, pl.ds(0, _L)] = zv
        zbuf[i, pl.ds(_L, _L)] = zv
        return 0
    lax.fori_loop(0, _ZR, zrow, 0)

    def ztmp(t, _):
        degb[pl.ds(t * _L, _L)] = zv
        return 0
    lax.fori_loop(0, _RPT // _L, ztmp, 0)

    def orow(t, _):
        onesb[pl.ds(t * _L, _L)] = jnp.ones((_L,), jnp.float32)
        return 0
    lax.fori_loop(0, _CHUNK // _L, orow, 0)

    pltpu.sync_copy(degb.at[pl.ds(0, _RPT)], deg_s.at[pl.ds(r0, _RPT)])

    def zs(c, _):
        pltpu.sync_copy(zbuf, s_s.at[pl.ds(r0 + c * _ZR, _ZR)])
        return 0
    lax.fori_loop(0, _RPT // _ZR, zs, 0)
    plsc.subcore_barrier()

    def degch(j, _):
        pltpu.async_copy(onesb, deg_s.at[coli.at[j]], ssem, add=True)
        return 0
    lax.fori_loop(0, _CH, degch, 0)

    def degdr(j, _):
        pltpu.make_async_copy(onesb, deg_s.at[coli.at[0]], ssem).wait()
        return 0
    lax.fori_loop(0, _CH, degdr, 0)
    plsc.subcore_barrier()

    pltpu.sync_copy(deg_s.at[pl.ds(r0, _RPT)], degb.at[pl.ds(0, _RPT)])

    def disrow(t, _):
        d = degb[pl.ds(t * _L, _L)] + jnp.float32(1.0)
        y = _nrsqrt(d)
        degb[pl.ds(t * _L, _L)] = y
        d2b[pl.ds(t * _L, _L)] = jnp.float32(0.9) / d
        idb[pl.ds(t * _L, _L)] = d * y
        return 0
    lax.fori_loop(0, _RPT // _L, disrow, 0)

    def initch(c, _):
        pltpu.sync_copy(
            h0s.at[cid, pl.ds(r0 + c * _CHUNK, _CHUNK)], buf.at[0])

        def initrow(i, _):
            r = c * _CHUNK + i
            dis = _sload(degb, r)
            v0 = buf[0, i, pl.ds(0, _L)] * dis
            v1 = buf[0, i, pl.ds(_L, _L)] * dis
            ga[r, pl.ds(0, _L)] = v0
            ga[r, pl.ds(_L, _L)] = v1
            g0p[r, pl.ds(0, _L)] = v0 * jnp.float32(0.1)
            g0p[r, pl.ds(_L, _L)] = v1 * jnp.float32(0.1)
            return 0
        lax.fori_loop(0, _CHUNK, initrow, 0)
        return 0
    lax.fori_loop(0, _CPB, initch, 0)
    pltpu.sync_copy(ga, g_s.at[pl.ds(r0, _RPT)])
    plsc.subcore_barrier()

    def wait_g():
        pltpu.make_async_copy(g_s.at[rowi.at[0]], buf.at[0], gsem).wait()

    def wait_s():
        pltpu.make_async_copy(buf.at[0], s_s.at[coli.at[0]], ssem).wait()

    def kstep(k, _):
        pltpu.async_copy(g_s.at[rowi.at[0]], buf.at[0], gsem)
        pltpu.async_copy(g_s.at[rowi.at[1]], buf.at[1], gsem)
        pltpu.async_copy(g_s.at[rowi.at[2]], buf.at[2], gsem)
        wait_g()
        pltpu.async_copy(buf.at[0], s_s.at[coli.at[0]], ssem, add=True)
        pltpu.async_copy(g_s.at[rowi.at[3]], buf.at[3], gsem)

        def ech(j, _):
            wait_g()
            pltpu.async_copy(
                buf.at[j % 4], s_s.at[coli.at[j]], ssem, add=True)
            wait_s()
            pltpu.async_copy(
                g_s.at[rowi.at[j + 3]], buf.at[(j + 3) % 4], gsem)
            return 0
        lax.fori_loop(1, _CH - 3, ech, 0)

        def etail(j, _):
            wait_g()
            pltpu.async_copy(
                buf.at[j % 4], s_s.at[coli.at[j]], ssem, add=True)
            wait_s()
            return 0
        lax.fori_loop(_CH - 3, _CH, etail, 0)
        wait_s()
        plsc.subcore_barrier()

        pltpu.async_copy(s_s.at[pl.ds(r0, _CHUNK)], buf.at[0], gsem)

        def dchunk(c, _):
            base = c * _CHUNK
            pltpu.make_async_copy(
                s_s.at[pl.ds(r0, _CHUNK)], buf.at[0], gsem).wait()

            @pl.when(c + 1 < _CPB)
            def _():
                pltpu.async_copy(
                    s_s.at[pl.ds(r0 + base + _CHUNK, _CHUNK)],
                    buf.at[(c + 1) % 3], gsem)
            pltpu.async_copy(zbuf, s_s.at[pl.ds(r0 + base, _ZR)], ssem)
            pltpu.async_copy(
                zbuf, s_s.at[pl.ds(r0 + base + _ZR, _ZR)], ssem)

            def drow(i, _):
                r = base + i
                d2 = _sload(d2b, r)
                v0 = (buf[c % 3, i, pl.ds(0, _L)] + ga[r, pl.ds(0, _L)]) \
                    * d2 + g0p[r, pl.ds(0, _L)]
                v1 = (buf[c % 3, i, pl.ds(_L, _L)] + ga[r, pl.ds(_L, _L)]) \
                    * d2 + g0p[r, pl.ds(_L, _L)]
                ga[r, pl.ds(0, _L)] = v0
                ga[r, pl.ds(_L, _L)] = v1
                return 0
            lax.fori_loop(0, _CHUNK, drow, 0)
            pltpu.async_copy(
                ga.at[pl.ds(base, _CHUNK)],
                g_s.at[pl.ds(r0 + base, _CHUNK)], ssem)
            return 0
        lax.fori_loop(0, _CPB, dchunk, 0)

        def ddr(c, _):
            pltpu.make_async_copy(zbuf, s_s.at[pl.ds(r0, _ZR)], ssem).wait()
            pltpu.make_async_copy(zbuf, s_s.at[pl.ds(r0, _ZR)], ssem).wait()
            pltpu.make_async_copy(
                ga.at[pl.ds(0, _CHUNK)],
                g_s.at[pl.ds(r0, _CHUNK)], ssem).wait()
            return 0
        lax.fori_loop(0, _CPB, ddr, 0)
        plsc.subcore_barrier()
        return 0
    lax.fori_loop(0, _K, kstep, 0)

    def bnrow(i, carry):
        s0, s1, q0, q1 = carry
        inv = _sload(idb, i)
        h0v = ga[i, pl.ds(0, _L)] * inv
        h1v = ga[i, pl.ds(_L, _L)] * inv
        return (s0 + h0v, s1 + h1v, q0 + h0v * h0v, q1 + h1v * h1v)
    s0, s1, q0, q1 = lax.fori_loop(0, _RPT, bnrow, (zv, zv, zv, zv))
    bnw[pl.ds(0, _L)] = s0
    bnw[pl.ds(_L, _L)] = s1
    bnw[pl.ds(2 * _L, _L)] = q0
    bnw[pl.ds(3 * _L, _L)] = q1
    pltpu.sync_copy(bnw, bn_s.at[sid])
    plsc.subcore_barrier()
    pltpu.sync_copy(bn_s, bnl)

    def red(t, carry):
        s0, s1, q0, q1 = carry
        return (s0 + bnl[t, pl.ds(0, _L)],
                s1 + bnl[t, pl.ds(_L, _L)],
                q0 + bnl[t, pl.ds(2 * _L, _L)],
                q1 + bnl[t, pl.ds(3 * _L, _L)])
    s0, s1, q0, q1 = lax.fori_loop(0, _NS, red, (zv, zv, zv, zv))
    ninv = jnp.float32(1.0 / _N)
    m0 = s0 * ninv
    m1 = s1 * ninv
    v0 = q0 * ninv - m0 * m0
    v1 = q1 * ninv - m1 * m1
    sc0 = gbl[0, pl.ds(0, _L)] * _nrsqrt(v0 + jnp.float32(_EPS))
    sc1 = gbl[0, pl.ds(_L, _L)] * _nrsqrt(v1 + jnp.float32(_EPS))
    sh0 = gbl[1, pl.ds(0, _L)] - sc0 * m0
    sh1 = gbl[1, pl.ds(_L, _L)] - sc1 * m1

    def arow(i, _):
        inv = _sload(idb, i)
        ga[i, pl.ds(0, _L)] = (ga[i, pl.ds(0, _L)] * inv) * sc0 + sh0
        ga[i, pl.ds(_L, _L)] = (ga[i, pl.ds(_L, _L)] * inv) * sc1 + sh1
        return 0
    lax.fori_loop(0, _RPT, arow, 0)
    c0 = cid * _DH

    @pl.when(sid < _NS - 1)
    def _():
        pltpu.sync_copy(
            ga.at[pl.ds(0, _RPT)],
            out_h.at[pl.ds(r0, _RPT), pl.ds(c0, _DH)])

    @pl.when(sid == _NS - 1)
    def _():
        pltpu.sync_copy(
            ga.at[pl.ds(0, _LAST)],
            out_h.at[pl.ds(r0, _LAST), pl.ds(c0, _DH)])


def kernel(x, edge_index, W, b, gamma, beta):
    f32 = jnp.float32
    x_pad = jnp.pad(x, ((0, _NPAD - _N), (0, 0)))
    b2 = b.reshape(_NC, 1, _DH)

    h0s = pl.pallas_call(
        _mm_body,
        grid=(_NPAD // _MMBLK, _NC),
        in_specs=[
            pl.BlockSpec((_MMBLK, _D_IN), lambda i, c: (i, 0)),
            pl.BlockSpec((_DH, _D_IN), lambda i, c: (c, 0)),
            pl.BlockSpec((1, 1, _DH), lambda i, c: (c, 0, 0)),
        ],
        out_specs=pl.BlockSpec((1, _MMBLK, _DH), lambda i, c: (c, i, 0)),
        out_shape=jax.ShapeDtypeStruct((_NC, _NPAD, _DH), f32),
    )(x_pad, W, b2)

    padn = _EPAD - _E
    pr = _N + (jnp.arange(padn, dtype=jnp.int32) % 112)
    pc = _N + 112 + (jnp.arange(padn, dtype=jnp.int32) % 112)
    rows_p = jnp.concatenate([edge_index[0], pr]).reshape(_NS, _CH, _CHUNK)
    cols_p = jnp.concatenate([edge_index[1], pc]).reshape(_NS, _CH, _CHUNK)
    gb = jnp.stack([gamma.reshape(_NC, _DH), beta.reshape(_NC, _DH)], axis=1)

    mesh = plsc.VectorSubcoreMesh(core_axis_name="c", subcore_axis_name="s")
    out = pl.kernel(
        _sc_body,
        out_type=jax.ShapeDtypeStruct((_N, _D_OUT), f32),
        mesh=mesh,
        compiler_params=pltpu.CompilerParams(use_tc_tiling_on_sc=False),
        scratch_types=[
            pltpu.VMEM_SHARED((_NPAD, _DH), f32),
            pltpu.VMEM_SHARED((_NPAD, _DH), f32),
            pltpu.VMEM_SHARED((_NPAD,), f32),
            pltpu.VMEM_SHARED((_NS, 4 * _L), f32),
            pltpu.VMEM((_CH, _CHUNK), jnp.int32),
            pltpu.VMEM((_CH, _CHUNK), jnp.int32),
            pltpu.VMEM((4, _CHUNK, _DH), f32),
            pltpu.VMEM((_ZR, _DH), f32),
            pltpu.VMEM((_RPT, _DH), f32),
            pltpu.VMEM((_RPT, _DH), f32),
            pltpu.VMEM((_RPT + _L,), f32),
            pltpu.VMEM((_RPT + _L,), f32),
            pltpu.VMEM((_RPT + _L,), f32),
            pltpu.VMEM((_CHUNK,), f32),
            pltpu.VMEM((4 * _L,), f32),
            pltpu.VMEM((_NS, 4 * _L), f32),
            pltpu.VMEM((2, _DH), f32),
            pltpu.SemaphoreType.DMA,
            pltpu.SemaphoreType.DMA,
        ],
    )(h0s, rows_p, cols_p, gb)

    return out


# speedup vs baseline: 35.3086x; 1.0113x over previous
import jax
import jax.numpy as jnp
from jax import lax
from jax.experimental import pallas as pl
from jax.experimental.pallas import tpu as pltpu
from jax.experimental.pallas import tpu_sc as plsc

_N = 10000
_E = 160000
_D_IN = 256
_D_OUT = 64
_K = 10
_EPS = 1e-5

_NC = 2
_NS = 16
_L = 16
_NPAD = 10240
_RPT = _NPAD // _NS
_DH = _D_OUT // _NC
_ECH = 125
_CH = 80
_CHUNK = 128

_MMBLK = 2048


def _mm_body(x_ref, w_ref, b_ref, o_ref):
    i = pl.program_id(0)
    acc = lax.dot_general(
        x_ref[...], w_ref[...], (((1,), (1,)), ((), ())),
        preferred_element_type=jnp.float32)
    rowid = i * _MMBLK + lax.broadcasted_iota(jnp.int32, (_MMBLK, _DH), 0)
    o_ref[...] = jnp.where(rowid < _N, acc + b_ref[0], 0.0)[None]


def _nrsqrt(x):
    i = lax.bitcast_convert_type(x, jnp.int32)
    i = jnp.int32(0x5F3759DF) - (i >> 1)
    y = lax.bitcast_convert_type(i, jnp.float32)
    for _ in range(3):
        y = y * (jnp.float32(1.5) - jnp.float32(0.5) * x * y * y)
    return y


def _eb(buf, b):
    return buf.at[b, pl.ds(0, _ECH)]


def _sload(ref, i):
    return ref[pl.ds(i, _L)][0]


_CPB = _RPT // _CHUNK
_ZR = 32
_LAST = _N - (_NS - 1) * _RPT


def _sc_body(h0s, rows_h, cols_h, gb_h, out_h,
             g_s, s_s, deg_s, bn_s,
             rowi, coli, buf, zbuf, ga, g0p, degb, d2b, idb, onesb,
             bnw, bnl, gbl, gsem, ssem):
    cid = lax.axis_index("c")
    sid = lax.axis_index("s")
    r0 = sid * _RPT

    pltpu.sync_copy(rows_h.at[sid], rowi)
    pltpu.sync_copy(cols_h.at[sid], coli)
    pltpu.sync_copy(gb_h.at[cid], gbl)

    zv = jnp.zeros((_L,), jnp.float32)

    def zrow(i, _):
        zbuf[i, pl.ds(0, _L)] = zv
        zbuf[i, pl.ds(_L, _L)] = zv
        return 0
    lax.fori_loop(0, _ZR, zrow, 0)

    def ztmp(t, _):
        degb[pl.ds(t * _L, _L)] = zv
        return 0
    lax.fori_loop(0, _RPT // _L, ztmp, 0)

    def orow(t, _):
        onesb[pl.ds(t * _L, _L)] = jnp.ones((_L,), jnp.float32)
        return 0
    lax.fori_loop(0, 8, orow, 0)

    pltpu.sync_copy(degb.at[pl.ds(0, _RPT)], deg_s.at[pl.ds(r0, _RPT)])

    def zs(c, _):
        pltpu.sync_copy(zbuf, s_s.at[pl.ds(r0 + c * _ZR, _ZR)])
        return 0
    lax.fori_loop(0, _RPT // _ZR, zs, 0)
    plsc.subcore_barrier()

    def degch(j, _):
        pltpu.async_copy(
            onesb.at[pl.ds(0, _ECH)], deg_s.at[coli.at[j]], ssem, add=True)
        return 0
    lax.fori_loop(0, _CH, degch, 0)

    def degdr(j, _):
        pltpu.make_async_copy(
            onesb.at[pl.ds(0, _ECH)], deg_s.at[coli.at[0]], ssem).wait()
        return 0
    lax.fori_loop(0, _CH, degdr, 0)
    plsc.subcore_barrier()

    pltpu.sync_copy(deg_s.at[pl.ds(r0, _RPT)], degb.at[pl.ds(0, _RPT)])

    def disrow(t, _):
        d = degb[pl.ds(t * _L, _L)] + jnp.float32(1.0)
        y = _nrsqrt(d)
        degb[pl.ds(t * _L, _L)] = y
        d2b[pl.ds(t * _L, _L)] = jnp.float32(0.9) / d
        idb[pl.ds(t * _L, _L)] = d * y
        return 0
    lax.fori_loop(0, _RPT // _L, disrow, 0)

    def initch(c, _):
        pltpu.sync_copy(
            h0s.at[cid, pl.ds(r0 + c * _CHUNK, _CHUNK)], buf.at[0])

        def initrow(i, _):
            r = c * _CHUNK + i
            dis = _sload(degb, r)
            v0 = buf[0, i, pl.ds(0, _L)] * dis
            v1 = buf[0, i, pl.ds(_L, _L)] * dis
            ga[r, pl.ds(0, _L)] = v0
            ga[r, pl.ds(_L, _L)] = v1
            g0p[r, pl.ds(0, _L)] = v0 * jnp.float32(0.1)
            g0p[r, pl.ds(_L, _L)] = v1 * jnp.float32(0.1)
            return 0
        lax.fori_loop(0, _CHUNK, initrow, 0)
        return 0
    lax.fori_loop(0, _CPB, initch, 0)
    pltpu.sync_copy(ga, g_s.at[pl.ds(r0, _RPT)])
    plsc.subcore_barrier()

    def wait_g():
        pltpu.make_async_copy(
            g_s.at[rowi.at[0]], _eb(buf, 0), gsem).wait()

    def wait_s():
        pltpu.make_async_copy(
            _eb(buf, 0), s_s.at[coli.at[0]], ssem).wait()

    def kstep(k, _):
        pltpu.async_copy(g_s.at[rowi.at[0]], _eb(buf, 0), gsem)
        pltpu.async_copy(g_s.at[rowi.at[1]], _eb(buf, 1), gsem)
        pltpu.async_copy(g_s.at[rowi.at[2]], _eb(buf, 2), gsem)
        wait_g()
        pltpu.async_copy(_eb(buf, 0), s_s.at[coli.at[0]], ssem, add=True)
        pltpu.async_copy(g_s.at[rowi.at[3]], _eb(buf, 3), gsem)

        def ech(j, _):
            wait_g()
            pltpu.async_copy(
                _eb(buf, j % 4), s_s.at[coli.at[j]], ssem, add=True)
            wait_s()
            pltpu.async_copy(
                g_s.at[rowi.at[j + 3]], _eb(buf, (j + 3) % 4), gsem)
            return 0
        lax.fori_loop(1, _CH - 3, ech, 0)

        def etail(j, _):
            wait_g()
            pltpu.async_copy(
                _eb(buf, j % 4), s_s.at[coli.at[j]], ssem, add=True)
            wait_s()
            return 0
        lax.fori_loop(_CH - 3, _CH, etail, 0)
        wait_s()
        plsc.subcore_barrier()

        pltpu.async_copy(s_s.at[pl.ds(r0, _CHUNK)], buf.at[0], gsem)

        def dchunk(c, _):
            base = c * _CHUNK
            pltpu.make_async_copy(
                s_s.at[pl.ds(r0, _CHUNK)], buf.at[0], gsem).wait()

            @pl.when(c + 1 < _CPB)
            def _():
                pltpu.async_copy(
                    s_s.at[pl.ds(r0 + base + _CHUNK, _CHUNK)],
                    buf.at[(c + 1) % 4], gsem)
            pltpu.async_copy(zbuf, s_s.at[pl.ds(r0 + base, _ZR)], ssem)
            pltpu.async_copy(
                zbuf, s_s.at[pl.ds(r0 + base + _ZR, _ZR)], ssem)
            pltpu.async_copy(
                zbuf, s_s.at[pl.ds(r0 + base + 2 * _ZR, _ZR)], ssem)
            pltpu.async_copy(
                zbuf, s_s.at[pl.ds(r0 + base + 3 * _ZR, _ZR)], ssem)

            def drow(i, _):
                r = base + i
                d2 = _sload(d2b, r)
                v0 = (buf[c % 4, i, pl.ds(0, _L)] + ga[r, pl.ds(0, _L)]) \
                    * d2 + g0p[r, pl.ds(0, _L)]
                v1 = (buf[c % 4, i, pl.ds(_L, _L)] + ga[r, pl.ds(_L, _L)]) \
                    * d2 + g0p[r, pl.ds(_L, _L)]
                ga[r, pl.ds(0, _L)] = v0
                ga[r, pl.ds(_L, _L)] = v1
                return 0
            lax.fori_loop(0, _CHUNK, drow, 0)
            pltpu.async_copy(
                ga.at[pl.ds(base, _CHUNK)],
                g_s.at[pl.ds(r0 + base, _CHUNK)], ssem)
            return 0
        lax.fori_loop(0, _CPB, dchunk, 0)

        def ddr(c, _):
            pltpu.make_async_copy(zbuf, s_s.at[pl.ds(r0, _ZR)], ssem).wait()
            pltpu.make_async_copy(zbuf, s_s.at[pl.ds(r0, _ZR)], ssem).wait()
            pltpu.make_async_copy(zbuf, s_s.at[pl.ds(r0, _ZR)], ssem).wait()
            pltpu.make_async_copy(zbuf, s_s.at[pl.ds(r0, _ZR)], ssem).wait()
            pltpu.make_async_copy(
                ga.at[pl.ds(0, _CHUNK)],
                g_s.at[pl.ds(r0, _CHUNK)], ssem).wait()
            return 0
        lax.fori_loop(0, _CPB, ddr, 0)
        plsc.subcore_barrier()
        return 0
    lax.fori_loop(0, _K, kstep, 0)

    def bnrow(i, carry):
        s0, s1, q0, q1 = carry
        inv = _sload(idb, i)
        h0v = ga[i, pl.ds(0, _L)] * inv
        h1v = ga[i, pl.ds(_L, _L)] * inv
        return (s0 + h0v, s1 + h1v, q0 + h0v * h0v, q1 + h1v * h1v)
    s0, s1, q0, q1 = lax.fori_loop(0, _RPT, bnrow, (zv, zv, zv, zv))
    bnw[pl.ds(0, _L)] = s0
    bnw[pl.ds(_L, _L)] = s1
    bnw[pl.ds(2 * _L, _L)] = q0
    bnw[pl.ds(3 * _L, _L)] = q1
    pltpu.sync_copy(bnw, bn_s.at[sid])
    plsc.subcore_barrier()
    pltpu.sync_copy(bn_s, bnl)

    def red(t, carry):
        s0, s1, q0, q1 = carry
        return (s0 + bnl[t, pl.ds(0, _L)],
                s1 + bnl[t, pl.ds(_L, _L)],
                q0 + bnl[t, pl.ds(2 * _L, _L)],
                q1 + bnl[t, pl.ds(3 * _L, _L)])
    s0, s1, q0, q1 = lax.fori_loop(0, _NS, red, (zv, zv, zv, zv))
    ninv = jnp.float32(1.0 / _N)
    m0 = s0 * ninv
    m1 = s1 * ninv
    v0 = q0 * ninv - m0 * m0
    v1 = q1 * ninv - m1 * m1
    sc0 = gbl[0, pl.ds(0, _L)] * _nrsqrt(v0 + jnp.float32(_EPS))
    sc1 = gbl[0, pl.ds(_L, _L)] * _nrsqrt(v1 + jnp.float32(_EPS))
    sh0 = gbl[1, pl.ds(0, _L)] - sc0 * m0
    sh1 = gbl[1, pl.ds(_L, _L)] - sc1 * m1

    def arow(i, _):
        inv = _sload(idb, i)
        ga[i, pl.ds(0, _L)] = (ga[i, pl.ds(0, _L)] * inv) * sc0 + sh0
        ga[i, pl.ds(_L, _L)] = (ga[i, pl.ds(_L, _L)] * inv) * sc1 + sh1
        return 0
    lax.fori_loop(0, _RPT, arow, 0)
    c0 = cid * _DH

    @pl.when(sid < _NS - 1)
    def _():
        pltpu.sync_copy(
            ga.at[pl.ds(0, _RPT)],
            out_h.at[pl.ds(r0, _RPT), pl.ds(c0, _DH)])

    @pl.when(sid == _NS - 1)
    def _():
        pltpu.sync_copy(
            ga.at[pl.ds(0, _LAST)],
            out_h.at[pl.ds(r0, _LAST), pl.ds(c0, _DH)])


def kernel(x, edge_index, W, b, gamma, beta):
    f32 = jnp.float32
    b2 = b.reshape(_NC, 1, _DH)

    h0s = pl.pallas_call(
        _mm_body,
        grid=(_NPAD // _MMBLK, _NC),
        in_specs=[
            pl.BlockSpec((_MMBLK, _D_IN), lambda i, c: (i, 0)),
            pl.BlockSpec((_DH, _D_IN), lambda i, c: (c, 0)),
            pl.BlockSpec((1, 1, _DH), lambda i, c: (c, 0, 0)),
        ],
        out_specs=pl.BlockSpec((1, _MMBLK, _DH), lambda i, c: (c, i, 0)),
        out_shape=jax.ShapeDtypeStruct((_NC, _NPAD, _DH), f32),
    )(x, W, b2)

    rows_p = edge_index[0].reshape(_NS, _CH, _ECH)
    cols_p = edge_index[1].reshape(_NS, _CH, _ECH)
    gb = jnp.stack([gamma.reshape(_NC, _DH), beta.reshape(_NC, _DH)], axis=1)

    mesh = plsc.VectorSubcoreMesh(core_axis_name="c", subcore_axis_name="s")
    out = pl.kernel(
        _sc_body,
        out_type=jax.ShapeDtypeStruct((_N, _D_OUT), f32),
        mesh=mesh,
        compiler_params=pltpu.CompilerParams(use_tc_tiling_on_sc=False),
        scratch_types=[
            pltpu.VMEM_SHARED((_NPAD, _DH), f32),
            pltpu.VMEM_SHARED((_NPAD, _DH), f32),
            pltpu.VMEM_SHARED((_NPAD,), f32),
            pltpu.VMEM_SHARED((_NS, 4 * _L), f32),
            pltpu.VMEM((_CH, _ECH), jnp.int32),
            pltpu.VMEM((_CH, _ECH), jnp.int32),
            pltpu.VMEM((4, _CHUNK, _DH), f32),
            pltpu.VMEM((_ZR, _DH), f32),
            pltpu.VMEM((_RPT, _DH), f32),
            pltpu.VMEM((_RPT, _DH), f32),
            pltpu.VMEM((_RPT + _L,), f32),
            pltpu.VMEM((_RPT + _L,), f32),
            pltpu.VMEM((_RPT + _L,), f32),
            pltpu.VMEM((_CHUNK,), f32),
            pltpu.VMEM((4 * _L,), f32),
            pltpu.VMEM((_NS, 4 * _L), f32),
            pltpu.VMEM((2, _DH), f32),
            pltpu.SemaphoreType.DMA,
            pltpu.SemaphoreType.DMA,
        ],
    )(h0s, rows_p, cols_p, gb)

    return out
